# baseline (speedup 1.0000x reference)
import functools
import jax
import jax.numpy as jnp
from jax import lax
from jax.experimental import pallas as pl
from jax.experimental.pallas import tpu as pltpu
from jax.experimental.pallas import tpu_sc as plsc

_VSIZE = 100000
_CTXLEN = 2048
_DMODEL = 1024
_B = 4
_T = 2048

_NC = 2
_NS = 16
_NW = _NC * _NS
_TPW = _T // _NW
_CHUNK = 32
_NCH = _TPW // _CHUNK
_LANES = 16
_VPR = _DMODEL // _LANES


def _emb_body(x_hbm, tok_hbm, pos_hbm, out_hbm, idx_v, rows_v, pos_v, sem):
    wid = lax.axis_index("s") * _NC + lax.axis_index("c")
    tb = wid * _TPW
    pltpu.sync_copy(pos_hbm.at[pl.ds(tb, _TPW)], pos_v)
    for b in range(_B):
        for c in range(_NCH):
            base = b * _T + tb + c * _CHUNK
            pltpu.sync_copy(x_hbm.at[pl.ds(base, _CHUNK)], idx_v)
            pltpu.async_copy(tok_hbm.at[idx_v], rows_v, sem).wait()

            def add_col(j, carry, c=c):
                col = pl.ds(j * _LANES, _LANES)
                for r in range(_CHUNK):
                    rows_v[r, col] = rows_v[r, col] + pos_v[c * _CHUNK + r, col]
                return carry

            lax.fori_loop(0, _VPR, add_col, 0)
            pltpu.sync_copy(rows_v, out_hbm.at[pl.ds(base, _CHUNK)])


@jax.jit
def kernel(x, tokemb, posemb):
    b, t = x.shape
    xf = x.reshape(b * t).astype(jnp.int32)
    mesh = plsc.VectorSubcoreMesh(core_axis_name="c", subcore_axis_name="s")
    out = pl.kernel(
        _emb_body,
        out_type=jax.ShapeDtypeStruct((b * t, _DMODEL), jnp.float32),
        mesh=mesh,
        scratch_types=[
            pltpu.VMEM((_CHUNK,), jnp.int32),
            pltpu.VMEM((_CHUNK, _DMODEL), jnp.float32),
            pltpu.VMEM((_TPW, _DMODEL), jnp.float32),
            pltpu.SemaphoreType.DMA,
        ],
    )(xf, tokemb, posemb)
    return out.reshape(b, t, _DMODEL)


# speedup vs baseline: 1.3141x; 1.3141x over previous
import jax
import jax.numpy as jnp
from jax import lax
from jax.experimental import pallas as pl
from jax.experimental.pallas import tpu as pltpu
from jax.experimental.pallas import tpu_sc as plsc

_DMODEL = 1024
_B = 4
_T = 2048

_NC = 2
_NS = 16
_NW = _NC * _NS
_TPW = _T // _NW
_CHUNK = 16
_NCH = _TPW // _CHUNK
_NBUF = 3
_LANES = 16
_VPR = _DMODEL // _LANES
_NIT = _NCH * _B


def _emb_body(x_hbm, tok_hbm, pos_hbm, out_hbm,
              idx_v, pos_v, rows_v, gsems, wsems, psem):
    wid = lax.axis_index("s") * _NC + lax.axis_index("c")
    tb = wid * _TPW

    pos_cp = pltpu.async_copy(pos_hbm.at[pl.ds(tb, _TPW)], pos_v, psem)
    for b in range(_B):
        pltpu.sync_copy(x_hbm.at[pl.ds(b * _T + tb, _TPW)], idx_v.at[b])

    def chunk_cb(i):
        return i // _B, i % _B

    def start_gather(i, p):
        c, b = chunk_cb(i)
        return pltpu.async_copy(
            tok_hbm.at[idx_v.at[b, pl.ds(c * _CHUNK, _CHUNK)]],
            rows_v[p], gsems[p])

    def start_write(i, p):
        c, b = chunk_cb(i)
        base = b * _T + tb + c * _CHUNK
        return pltpu.async_copy(rows_v[p], out_hbm.at[pl.ds(base, _CHUNK)],
                                wsems[p])

    g = [None] * _NBUF
    w = [None] * _NBUF
    for j in range(_NBUF - 1):
        g[j] = start_gather(j, j)
    pos_cp.wait()

    for i in range(_NIT):
        p = i % _NBUF
        nxt = i + _NBUF - 1
        if nxt < _NIT:
            q = nxt % _NBUF
            if w[q] is not None:
                w[q].wait()
                w[q] = None
            g[q] = start_gather(nxt, q)
        g[p].wait()

        c, _ = chunk_cb(i)
        buf = rows_v[p]

        def add_col(j, carry, c=c, buf=buf):
            col = pl.ds(j * _LANES, _LANES)
            for r in range(_CHUNK):
                buf[r, col] = buf[r, col] + pos_v[c * _CHUNK + r, col]
            return carry

        lax.fori_loop(0, _VPR, add_col, 0)
        w[p] = start_write(i, p)

    for p in range(_NBUF):
        if w[p] is not None:
            w[p].wait()


@jax.jit
def kernel(x, tokemb, posemb):
    b, t = x.shape
    mesh = plsc.VectorSubcoreMesh(core_axis_name="c", subcore_axis_name="s")
    out = pl.kernel(
        _emb_body,
        out_type=jax.ShapeDtypeStruct((b * t, _DMODEL), jnp.float32),
        mesh=mesh,
        scratch_types=[
            pltpu.VMEM((_B, _TPW), jnp.int32),
            pltpu.VMEM((_TPW, _DMODEL), jnp.float32),
            [pltpu.VMEM((_CHUNK, _DMODEL), jnp.float32)] * _NBUF,
            [pltpu.SemaphoreType.DMA] * _NBUF,
            [pltpu.SemaphoreType.DMA] * _NBUF,
            pltpu.SemaphoreType.DMA,
        ],
    )(x.reshape(b * t).astype(jnp.int32), tokemb, posemb)
    return out.reshape(b, t, _DMODEL)
